# baseline (speedup 1.0000x reference)
import functools

import jax
import jax.numpy as jnp
from jax import lax
from jax.experimental import pallas as pl
from jax.experimental.pallas import tpu as pltpu
from jax.experimental.pallas import tpu_sc as plsc

N_LEVEL = 26
N_EMB = 100000
D_EMB = 32
BATCH = 16384

CHUNK = 1024
TOTAL = N_LEVEL * BATCH
N_CHUNKS = TOTAL // CHUNK
L = 16


def _body(idx_hbm, w_hbm, out_hbm, idx_v, rows_v, sem, *, chunks_per_w):
    wid = lax.axis_index("s") * 2 + lax.axis_index("c")

    def run_chunk(k, _):
        c = wid * chunks_per_w + k
        base = c * CHUNK
        offset = lax.shift_right_logical(base, 14) * N_EMB
        pltpu.sync_copy(idx_hbm.at[pl.ds(base, CHUNK)], idx_v)

        def add_off(i, _):
            sl = pl.ds(i * L, L)
            idx_v[sl] = idx_v[sl] + offset
            return 0

        lax.fori_loop(0, CHUNK // L, add_off, 0)
        pltpu.async_copy(w_hbm.at[idx_v], rows_v, sem).wait()
        pltpu.sync_copy(rows_v, out_hbm.at[pl.ds(base, CHUNK)])
        return 0

    lax.fori_loop(0, chunks_per_w, run_chunk, 0)


def kernel(idx, weight):
    info = plsc.get_sparse_core_info()
    nw = info.num_cores * info.num_subcores
    chunks_per_w = N_CHUNKS // nw

    idx_flat = idx.reshape(TOTAL).astype(jnp.int32)
    w_flat = weight.reshape(N_LEVEL * N_EMB, D_EMB)

    mesh = plsc.VectorSubcoreMesh(core_axis_name="c", subcore_axis_name="s")
    out = pl.kernel(
        functools.partial(_body, chunks_per_w=chunks_per_w),
        out_type=jax.ShapeDtypeStruct((TOTAL, D_EMB), jnp.float32),
        mesh=mesh,
        compiler_params=pltpu.CompilerParams(use_tc_tiling_on_sc=False),
        scratch_types=[
            pltpu.VMEM((CHUNK,), jnp.int32),
            pltpu.VMEM((CHUNK, D_EMB), jnp.float32),
            pltpu.SemaphoreType.DMA,
        ],
    )(idx_flat, w_flat)
    return out.reshape(N_LEVEL, BATCH, D_EMB)


# speedup vs baseline: 3.8522x; 3.8522x over previous
import jax
import jax.numpy as jnp
from jax import lax
from jax.experimental import pallas as pl
from jax.experimental.pallas import tpu as pltpu
from jax.experimental.pallas import tpu_sc as plsc

N_LEVEL = 26
N_EMB = 100000
D_EMB = 32
BATCH = 16384

BCHUNK = 4096
L = 16
GROUPS = BCHUNK // L
UNROLL = 8


def _body(idx_hbm, w_hbm, out_hbm, plane_v, idx_v, out_v):
    wid = lax.axis_index("s") * 2 + lax.axis_index("c")

    def do_level(l, _):
        r = l * D_EMB + wid
        pltpu.sync_copy(w_hbm.at[r], plane_v)

        def do_chunk(c, _):
            b0 = c * BCHUNK
            pltpu.sync_copy(idx_hbm.at[l, pl.ds(b0, BCHUNK)], idx_v)

            def do_groups(g, _):
                for u in range(UNROLL):
                    sl = pl.ds((g * UNROLL + u) * L, L)
                    out_v[sl] = plsc.load_gather(plane_v, [idx_v[sl]])
                return 0

            lax.fori_loop(0, GROUPS // UNROLL, do_groups, 0)
            pltpu.sync_copy(out_v, out_hbm.at[r, pl.ds(b0, BCHUNK)])
            return 0

        lax.fori_loop(0, BATCH // BCHUNK, do_chunk, 0)
        return 0

    lax.fori_loop(0, N_LEVEL, do_level, 0)


def kernel(idx, weight):
    w2 = jnp.swapaxes(weight, 1, 2).reshape(N_LEVEL * D_EMB, N_EMB)
    idx = idx.astype(jnp.int32)

    mesh = plsc.VectorSubcoreMesh(core_axis_name="c", subcore_axis_name="s")
    out_t = pl.kernel(
        _body,
        out_type=jax.ShapeDtypeStruct((N_LEVEL * D_EMB, BATCH), jnp.float32),
        mesh=mesh,
        compiler_params=pltpu.CompilerParams(needs_layout_passes=False),
        scratch_types=[
            pltpu.VMEM((N_EMB,), jnp.float32),
            pltpu.VMEM((BCHUNK,), jnp.int32),
            pltpu.VMEM((BCHUNK,), jnp.float32),
        ],
    )(idx, w2)
    return out_t.reshape(N_LEVEL, D_EMB, BATCH).swapaxes(1, 2)
